# baseline (speedup 1.0000x reference)
import functools

import jax
import jax.numpy as jnp
from jax import lax
from jax.experimental import pallas as pl
from jax.experimental.pallas import tpu as pltpu
from jax.experimental.pallas import tpu_sc as plsc

T = 4096
D = 256
N_PHONES = 64
K = 64
NC_TOTAL = N_PHONES * K

BT = 1024

_NC = 2
_NS = 16
_NW = _NC * _NS
_B_PER_W = T // _NW


def _argmin_body(h_ref, c_ref, ph_ref, cp_ref, cf_ref, idx_ref, ct_scr, cn_scr):
    i = pl.program_id(0)

    @pl.when(i == 0)
    def _init():
        c = c_ref[...]
        ct_scr[...] = jnp.swapaxes(c, 0, 1)
        cn_scr[...] = jnp.sum(ct_scr[...] * ct_scr[...], axis=0, keepdims=True)

    h_blk = h_ref[...]
    scores = lax.dot_general(
        h_blk, ct_scr[...], (((1,), (0,)), ((), ())),
        precision=lax.Precision.HIGHEST,
        preferred_element_type=jnp.float32,
    )
    phone = ph_ref[...]
    colphone = cp_ref[...]
    colf = cf_ref[...]
    dist = cn_scr[...] - 2.0 * scores + jnp.where(
        colphone == phone, 0.0, jnp.float32(1e9))
    m = jnp.min(dist, axis=1, keepdims=True)
    cand = jnp.where(dist == m, colf, jnp.float32(2e9))
    idx_ref[...] = jnp.min(cand, axis=1, keepdims=True).astype(jnp.int32)


def _tc_argmin(h, cflat, phones2d, colphone, colf, interpret=False):
    n_tok = h.shape[0]
    grid = n_tok // BT
    return pl.pallas_call(
        _argmin_body,
        grid=(grid,),
        in_specs=[
            pl.BlockSpec((BT, D), lambda i: (i, 0)),
            pl.BlockSpec((NC_TOTAL, D), lambda i: (0, 0)),
            pl.BlockSpec((BT, 1), lambda i: (i, 0)),
            pl.BlockSpec((1, NC_TOTAL), lambda i: (0, 0)),
            pl.BlockSpec((1, NC_TOTAL), lambda i: (0, 0)),
        ],
        out_specs=pl.BlockSpec((BT, 1), lambda i: (i, 0)),
        out_shape=jax.ShapeDtypeStruct((n_tok, 1), jnp.int32),
        scratch_shapes=[
            pltpu.VMEM((D, NC_TOTAL), jnp.float32),
            pltpu.VMEM((1, NC_TOTAL), jnp.float32),
        ],
        interpret=interpret,
    )(h, cflat, phones2d, colphone, colf)


@functools.cache
def _make_sc_gather(n_tok):
    b_per_w = n_tok // _NW
    @functools.partial(
        pl.kernel,
        mesh=plsc.VectorSubcoreMesh(core_axis_name="c", subcore_axis_name="s"),
        out_type=jax.ShapeDtypeStruct((n_tok, D), jnp.float32),
        scratch_types=[
            pltpu.VMEM((b_per_w,), jnp.int32),
            pltpu.VMEM((b_per_w, D), jnp.float32),
            pltpu.SemaphoreType.DMA,
        ],
    )
    def _sc_gather(table_hbm, idx_hbm, out_hbm, idx_v, rows_v, sem):
        wid = lax.axis_index("s") * _NC + lax.axis_index("c")
        base = wid * b_per_w
        pltpu.sync_copy(idx_hbm.at[pl.ds(base, b_per_w)], idx_v)
        pltpu.async_copy(table_hbm.at[idx_v], rows_v, sem).wait()
        pltpu.sync_copy(rows_v, out_hbm.at[pl.ds(base, b_per_w)])

    return _sc_gather


def kernel(h, phones, centers):
    cflat = centers.reshape(NC_TOTAL, D)
    phones2d = phones.astype(jnp.int32).reshape(T, 1)
    cols = jnp.arange(NC_TOTAL, dtype=jnp.int32).reshape(1, NC_TOTAL)
    colphone = cols >> 6
    colf = cols.astype(jnp.float32)
    half = T // 2
    gather = _make_sc_gather(half)
    idx1 = _tc_argmin(h[:half], cflat, phones2d[:half], colphone, colf)
    out1 = gather(cflat, idx1.reshape(half))
    idx2 = _tc_argmin(h[half:], cflat, phones2d[half:], colphone, colf)
    out2 = gather(cflat, idx2.reshape(half))
    return jnp.concatenate([out1, out2], axis=0)


# speedup vs baseline: 1.1075x; 1.1075x over previous
import functools

import jax
import jax.numpy as jnp
from jax import lax
from jax.experimental import pallas as pl
from jax.experimental.pallas import tpu as pltpu
from jax.experimental.pallas import tpu_sc as plsc

T = 4096
D = 256
N_PHONES = 64
K = 64
NC_TOTAL = N_PHONES * K

BT = 2048

_NC = 2
_NS = 16
_NW = _NC * _NS
_B_PER_W = T // _NW


def _argmin_body(h_ref, c_ref, ph_ref, cp_ref, cf_ref, idx_ref, ct_scr, cn_scr):
    i = pl.program_id(0)

    @pl.when(i == 0)
    def _init():
        c = c_ref[...]
        ct_scr[...] = jnp.swapaxes(c, 0, 1)
        cn_scr[...] = jnp.sum(ct_scr[...] * ct_scr[...], axis=0, keepdims=True)

    h_blk = h_ref[...]
    scores = lax.dot_general(
        h_blk, ct_scr[...], (((1,), (0,)), ((), ())),
        precision=lax.Precision.HIGHEST,
        preferred_element_type=jnp.float32,
    )
    phone = ph_ref[...]
    colphone = cp_ref[...]
    colf = cf_ref[...]
    dist = cn_scr[...] - 2.0 * scores + jnp.where(
        colphone == phone, 0.0, jnp.float32(1e9))
    m = jnp.min(dist, axis=1, keepdims=True)
    cand = jnp.where(dist == m, colf, jnp.float32(2e9))
    idx_ref[...] = jnp.min(cand, axis=1, keepdims=True).astype(jnp.int32)


def _tc_argmin(h, cflat, phones2d, colphone, colf, interpret=False):
    grid = T // BT
    return pl.pallas_call(
        _argmin_body,
        grid=(grid,),
        in_specs=[
            pl.BlockSpec((BT, D), lambda i: (i, 0)),
            pl.BlockSpec((NC_TOTAL, D), lambda i: (0, 0)),
            pl.BlockSpec((BT, 1), lambda i: (i, 0)),
            pl.BlockSpec((1, NC_TOTAL), lambda i: (0, 0)),
            pl.BlockSpec((1, NC_TOTAL), lambda i: (0, 0)),
        ],
        out_specs=pl.BlockSpec((BT, 1), lambda i: (i, 0)),
        out_shape=jax.ShapeDtypeStruct((T, 1), jnp.int32),
        scratch_shapes=[
            pltpu.VMEM((D, NC_TOTAL), jnp.float32),
            pltpu.VMEM((1, NC_TOTAL), jnp.float32),
        ],
        interpret=interpret,
    )(h, cflat, phones2d, colphone, colf)


@functools.cache
def _make_sc_gather():
    @functools.partial(
        pl.kernel,
        mesh=plsc.VectorSubcoreMesh(core_axis_name="c", subcore_axis_name="s"),
        out_type=jax.ShapeDtypeStruct((T, D), jnp.float32),
        scratch_types=[
            pltpu.VMEM((_B_PER_W,), jnp.int32),
            pltpu.VMEM((_B_PER_W, D), jnp.float32),
            pltpu.SemaphoreType.DMA,
        ],
    )
    def _sc_gather(table_hbm, idx_hbm, out_hbm, idx_v, rows_v, sem):
        wid = lax.axis_index("s") * _NC + lax.axis_index("c")
        base = wid * _B_PER_W
        pltpu.sync_copy(idx_hbm.at[pl.ds(base, _B_PER_W)], idx_v)
        pltpu.async_copy(table_hbm.at[idx_v], rows_v, sem).wait()
        pltpu.sync_copy(rows_v, out_hbm.at[pl.ds(base, _B_PER_W)])

    return _sc_gather


def kernel(h, phones, centers):
    cflat = centers.reshape(NC_TOTAL, D)
    phones2d = phones.astype(jnp.int32).reshape(T, 1)
    cols = jnp.arange(NC_TOTAL, dtype=jnp.int32).reshape(1, NC_TOTAL)
    colphone = cols >> 6
    colf = cols.astype(jnp.float32)
    idx = _tc_argmin(h, cflat, phones2d, colphone, colf)
    return _make_sc_gather()(cflat, idx.reshape(T))


# speedup vs baseline: 1.1587x; 1.0462x over previous
import functools

import jax
import jax.numpy as jnp
from jax import lax
from jax.experimental import pallas as pl
from jax.experimental.pallas import tpu as pltpu
from jax.experimental.pallas import tpu_sc as plsc

T = 4096
D = 256
N_PHONES = 64
K = 64
NC_TOTAL = N_PHONES * K

BT = 1024

_NC = 2
_NS = 16
_NW = _NC * _NS
_B_PER_W = T // _NW


def _argmin_body(h_ref, c_ref, ph_ref, cp_ref, cf_ref, idx_ref, ct_scr, cn_scr):
    i = pl.program_id(0)

    @pl.when(i == 0)
    def _init():
        c = c_ref[...]
        ct_scr[...] = jnp.swapaxes(c, 0, 1)
        cn_scr[...] = jnp.sum(ct_scr[...] * ct_scr[...], axis=0, keepdims=True)

    h2 = h_ref[...] * 2.0
    scores2 = lax.dot_general(
        h2, ct_scr[...], (((1,), (0,)), ((), ())),
        precision=lax.Precision.HIGHEST,
        preferred_element_type=jnp.float32,
    )
    phone = ph_ref[...]
    colphone = cp_ref[...]
    colf = cf_ref[...]
    pen_cn = jnp.where(colphone == phone, cn_scr[...], jnp.float32(1e9))
    dist = pen_cn - scores2
    m = jnp.min(dist, axis=1, keepdims=True)
    cand = jnp.where(dist == m, colf, jnp.float32(2e9))
    idx_ref[...] = jnp.min(cand, axis=1, keepdims=True).astype(jnp.int32)


def _tc_argmin(h, cflat, phones2d, colphone, colf, interpret=False):
    grid = T // BT
    return pl.pallas_call(
        _argmin_body,
        grid=(grid,),
        in_specs=[
            pl.BlockSpec((BT, D), lambda i: (i, 0)),
            pl.BlockSpec((NC_TOTAL, D), lambda i: (0, 0)),
            pl.BlockSpec((BT, 1), lambda i: (i, 0)),
            pl.BlockSpec((1, NC_TOTAL), lambda i: (0, 0)),
            pl.BlockSpec((1, NC_TOTAL), lambda i: (0, 0)),
        ],
        out_specs=pl.BlockSpec((BT, 1), lambda i: (i, 0)),
        out_shape=jax.ShapeDtypeStruct((T, 1), jnp.int32),
        scratch_shapes=[
            pltpu.VMEM((D, NC_TOTAL), jnp.float32),
            pltpu.VMEM((1, NC_TOTAL), jnp.float32),
        ],
        interpret=interpret,
    )(h, cflat, phones2d, colphone, colf)


@functools.cache
def _make_sc_gather():
    @functools.partial(
        pl.kernel,
        mesh=plsc.VectorSubcoreMesh(core_axis_name="c", subcore_axis_name="s"),
        out_type=jax.ShapeDtypeStruct((T, D), jnp.float32),
        scratch_types=[
            pltpu.VMEM((_B_PER_W,), jnp.int32),
            pltpu.VMEM((_B_PER_W, D), jnp.float32),
            pltpu.SemaphoreType.DMA,
        ],
    )
    def _sc_gather(table_hbm, idx_hbm, out_hbm, idx_v, rows_v, sem):
        wid = lax.axis_index("s") * _NC + lax.axis_index("c")
        base = wid * _B_PER_W
        pltpu.sync_copy(idx_hbm.at[pl.ds(base, _B_PER_W)], idx_v)
        pltpu.async_copy(table_hbm.at[idx_v], rows_v, sem).wait()
        pltpu.sync_copy(rows_v, out_hbm.at[pl.ds(base, _B_PER_W)])

    return _sc_gather


def kernel(h, phones, centers):
    cflat = centers.reshape(NC_TOTAL, D)
    phones2d = phones.astype(jnp.int32).reshape(T, 1)
    cols = jnp.arange(NC_TOTAL, dtype=jnp.int32).reshape(1, NC_TOTAL)
    colphone = cols >> 6
    colf = cols.astype(jnp.float32)
    idx = _tc_argmin(h, cflat, phones2d, colphone, colf)
    return _make_sc_gather()(cflat, idx.reshape(T))


# speedup vs baseline: 1.1602x; 1.0014x over previous
import functools

import jax
import jax.numpy as jnp
from jax import lax
from jax.experimental import pallas as pl
from jax.experimental.pallas import tpu as pltpu
from jax.experimental.pallas import tpu_sc as plsc

T = 4096
D = 256
N_PHONES = 64
K = 64
NC_TOTAL = N_PHONES * K

BT = 1024

_NC = 2
_NS = 16
_NW = _NC * _NS
_B_PER_W = T // _NW


def _argmin_body(h_ref, c_ref, ph_ref, cp_ref, cf_ref, idx_ref, ct_scr, cn_scr):
    i = pl.program_id(0)

    @pl.when(i == 0)
    def _init():
        c = c_ref[...]
        ct_scr[...] = jnp.swapaxes(c, 0, 1)
        cn_scr[...] = jnp.sum(ct_scr[...] * ct_scr[...], axis=0, keepdims=True)

    h2 = h_ref[...] * 2.0
    scores2 = lax.dot_general(
        h2, ct_scr[...], (((1,), (0,)), ((), ())),
        precision=lax.Precision.HIGHEST,
        preferred_element_type=jnp.float32,
    )
    phone = ph_ref[...]
    colphone = cp_ref[...]
    colf = cf_ref[...]
    pen_cn = jnp.where(colphone == phone, cn_scr[...], jnp.float32(1e9))
    dist = pen_cn - scores2
    m = jnp.min(dist, axis=1, keepdims=True)
    cand = jnp.where(dist == m, colf, jnp.float32(2e9))
    idx_ref[...] = jnp.min(cand, axis=1, keepdims=True).astype(jnp.int32)


def _tc_argmin(h, cflat, phones2d, colphone, colf):
    grid = T // BT
    return pl.pallas_call(
        _argmin_body,
        grid=(grid,),
        in_specs=[
            pl.BlockSpec((BT, D), lambda i: (i, 0)),
            pl.BlockSpec((NC_TOTAL, D), lambda i: (0, 0)),
            pl.BlockSpec((BT, 1), lambda i: (i, 0)),
            pl.BlockSpec((1, NC_TOTAL), lambda i: (0, 0)),
            pl.BlockSpec((1, NC_TOTAL), lambda i: (0, 0)),
        ],
        out_specs=pl.BlockSpec((BT, 1), lambda i: (i, 0)),
        out_shape=jax.ShapeDtypeStruct((T, 1), jnp.int32),
        scratch_shapes=[
            pltpu.VMEM((D, NC_TOTAL), jnp.float32),
            pltpu.VMEM((1, NC_TOTAL), jnp.float32),
        ],
    )(h, cflat, phones2d, colphone, colf)


@functools.cache
def _make_sc_gather():
    @functools.partial(
        pl.kernel,
        mesh=plsc.VectorSubcoreMesh(core_axis_name="c", subcore_axis_name="s"),
        out_type=jax.ShapeDtypeStruct((T, D), jnp.float32),
        scratch_types=[
            pltpu.VMEM((_B_PER_W,), jnp.int32),
            pltpu.VMEM((_B_PER_W, D), jnp.float32),
            pltpu.SemaphoreType.DMA,
        ],
    )
    def _sc_gather(table_hbm, idx_hbm, out_hbm, idx_v, rows_v, sem):
        wid = lax.axis_index("s") * _NC + lax.axis_index("c")
        base = wid * _B_PER_W
        pltpu.sync_copy(idx_hbm.at[pl.ds(base, _B_PER_W)], idx_v)
        pltpu.async_copy(table_hbm.at[idx_v], rows_v, sem).wait()
        pltpu.sync_copy(rows_v, out_hbm.at[pl.ds(base, _B_PER_W)])

    return _sc_gather


def kernel(h, phones, centers):
    cflat = centers.reshape(NC_TOTAL, D)
    phones2d = phones.astype(jnp.int32).reshape(T, 1)
    cols = jnp.arange(NC_TOTAL, dtype=jnp.int32).reshape(1, NC_TOTAL)
    colphone = cols >> 6
    colf = cols.astype(jnp.float32)
    idx = _tc_argmin(h, cflat, phones2d, colphone, colf)
    return _make_sc_gather()(cflat, idx.reshape(T))
